# baseline (speedup 1.0000x reference)
import jax
import jax.numpy as jnp
from jax import lax
from jax.experimental import pallas as pl
from jax.experimental.pallas import tpu as pltpu
from jax.experimental.pallas import tpu_sc as plsc

VOCAB_ = 1000000
DIM_ = 64
B_ = 16384
NNEG_ = 5
MARGIN_ = 1.0

NC = 2
NS = 16
NW = NC * NS
LANES = 16

BPW = B_ // NW
CHUNK = 128
NCHUNK = BPW // CHUNK
GROUPS = CHUNK // LANES


TW = 8192
NTB = (VOCAB_ + TW - 1) // TW
LIN = NTB * TW * DIM_


def _tc_transpose_body(x_ref, o_ref):
  t = x_ref[...].T
  t3 = t.reshape(TW // 2, 2, DIM_)
  o_ref[...] = jnp.concatenate([t3[:, 0, :], t3[:, 1, :]], axis=1)


def _to_rowmajor(table_t):
  out2d = pl.pallas_call(
      _tc_transpose_body,
      grid=(NTB,),
      in_specs=[pl.BlockSpec((DIM_, TW), lambda i: (0, i))],
      out_specs=pl.BlockSpec((TW // 2, 128), lambda i: (i, 0)),
      out_shape=jax.ShapeDtypeStruct((LIN // 128, 128), jnp.float32),
  )(table_t)
  return out2d.reshape(LIN)


def _shuf(x, perm):
  return lax.gather(
      x, perm[:, None],
      lax.GatherDimensionNumbers(offset_dims=(), collapsed_slice_dims=(0,),
                                 start_index_map=(0,)),
      slice_sizes=(1,), mode=lax.GatherScatterMode.PROMISE_IN_BOUNDS)


def _sc_body(u_hbm, v_hbm, n0_hbm, n1_hbm, n2_hbm, n3_hbm, n4_hbm,
             src_hbm, tgt_hbm, out_hbm,
             iu, iv, in0, in1, in2, in3, in4,
             ru, rv, rn0, rn1, rn2, rn3, rn4,
             acc_v, sem):
  cid = lax.axis_index("c")
  sid = lax.axis_index("s")
  wid = cid * NS + sid

  lane = lax.iota(jnp.int32, LANES)
  perms = [lane ^ 1, lane ^ 2, lane ^ 4, lane ^ 8]
  total = jnp.zeros((LANES,), jnp.float32)

  for chunk in range(NCHUNK):
    base = wid * BPW + chunk * CHUNK
    pltpu.sync_copy(u_hbm.at[pl.ds(base, CHUNK)], iu)
    pltpu.sync_copy(v_hbm.at[pl.ds(base, CHUNK)], iv)
    pltpu.sync_copy(n0_hbm.at[pl.ds(base, CHUNK)], in0)
    pltpu.sync_copy(n1_hbm.at[pl.ds(base, CHUNK)], in1)
    pltpu.sync_copy(n2_hbm.at[pl.ds(base, CHUNK)], in2)
    pltpu.sync_copy(n3_hbm.at[pl.ds(base, CHUNK)], in3)
    pltpu.sync_copy(n4_hbm.at[pl.ds(base, CHUNK)], in4)

    def row_dma(g, carry):
      gbase = g * LANES
      sl = pl.ds(gbase, LANES)
      vu, vv = iu[sl], iv[sl]
      v0, v1, v2, v3, v4 = in0[sl], in1[sl], in2[sl], in3[sl], in4[sl]
      for j in range(LANES):
        eb = (gbase + j) * DIM_
        pltpu.make_async_copy(src_hbm.at[pl.ds(vu[j] * DIM_, DIM_)],
                              ru.at[pl.ds(eb, DIM_)], sem).start()
        pltpu.make_async_copy(tgt_hbm.at[pl.ds(vv[j] * DIM_, DIM_)],
                              rv.at[pl.ds(eb, DIM_)], sem).start()
        pltpu.make_async_copy(tgt_hbm.at[pl.ds(v0[j] * DIM_, DIM_)],
                              rn0.at[pl.ds(eb, DIM_)], sem).start()
        pltpu.make_async_copy(tgt_hbm.at[pl.ds(v1[j] * DIM_, DIM_)],
                              rn1.at[pl.ds(eb, DIM_)], sem).start()
        pltpu.make_async_copy(tgt_hbm.at[pl.ds(v2[j] * DIM_, DIM_)],
                              rn2.at[pl.ds(eb, DIM_)], sem).start()
        pltpu.make_async_copy(tgt_hbm.at[pl.ds(v3[j] * DIM_, DIM_)],
                              rn3.at[pl.ds(eb, DIM_)], sem).start()
        pltpu.make_async_copy(tgt_hbm.at[pl.ds(v4[j] * DIM_, DIM_)],
                              rn4.at[pl.ds(eb, DIM_)], sem).start()
      return carry

    lax.fori_loop(0, GROUPS, row_dma, jnp.int32(0))
    for buf in (ru, rv, rn0, rn1, rn2, rn3, rn4):
      pltpu.make_async_copy(src_hbm.at[pl.ds(0, CHUNK * DIM_)], buf,
                            sem).wait()

    def elem_body(e, tot):
      pv = jnp.zeros((LANES,), jnp.float32)
      p0 = jnp.zeros((LANES,), jnp.float32)
      p1 = jnp.zeros((LANES,), jnp.float32)
      p2 = jnp.zeros((LANES,), jnp.float32)
      p3 = jnp.zeros((LANES,), jnp.float32)
      p4 = jnp.zeros((LANES,), jnp.float32)
      for k in range(DIM_ // LANES):
        sl = pl.ds(e * DIM_ + k * LANES, LANES)
        uc = ru[sl]
        pv = pv + uc * rv[sl]
        p0 = p0 + uc * rn0[sl]
        p1 = p1 + uc * rn1[sl]
        p2 = p2 + uc * rn2[sl]
        p3 = p3 + uc * rn3[sl]
        p4 = p4 + uc * rn4[sl]
      loss = jnp.zeros((LANES,), jnp.float32)
      for p in (p0, p1, p2, p3, p4):
        r = p - pv
        for perm in perms:
          r = r + _shuf(r, perm)
        loss = loss + jnp.maximum(r + MARGIN_, 0.0)
      return tot + loss

    total = total + lax.fori_loop(0, CHUNK, elem_body,
                                  jnp.zeros((LANES,), jnp.float32))

  acc_v[...] = jnp.where(lane == 0, total, jnp.float32(0.0))
  pltpu.sync_copy(acc_v, out_hbm.at[pl.ds(wid * LANES, LANES)])


@jax.jit
def _sc_call(u_pos, v_pos, n0, n1, n2, n3, n4, src_w, tgt_w):
  mesh = plsc.VectorSubcoreMesh(core_axis_name="c", subcore_axis_name="s")
  f = pl.kernel(
      _sc_body,
      out_type=jax.ShapeDtypeStruct((NW * LANES,), jnp.float32),
      mesh=mesh,
      scratch_types=[
          pltpu.VMEM((CHUNK,), jnp.int32),
          pltpu.VMEM((CHUNK,), jnp.int32),
          pltpu.VMEM((CHUNK,), jnp.int32),
          pltpu.VMEM((CHUNK,), jnp.int32),
          pltpu.VMEM((CHUNK,), jnp.int32),
          pltpu.VMEM((CHUNK,), jnp.int32),
          pltpu.VMEM((CHUNK,), jnp.int32),
          pltpu.VMEM((CHUNK * DIM_,), jnp.float32),
          pltpu.VMEM((CHUNK * DIM_,), jnp.float32),
          pltpu.VMEM((CHUNK * DIM_,), jnp.float32),
          pltpu.VMEM((CHUNK * DIM_,), jnp.float32),
          pltpu.VMEM((CHUNK * DIM_,), jnp.float32),
          pltpu.VMEM((CHUNK * DIM_,), jnp.float32),
          pltpu.VMEM((CHUNK * DIM_,), jnp.float32),
          pltpu.VMEM((LANES,), jnp.float32),
          pltpu.SemaphoreType.DMA,
      ],
  )
  return f(u_pos, v_pos, n0, n1, n2, n3, n4, src_w, tgt_w)


def kernel(u_pos, v_pos, v_neg, src_w, tgt_w):
  u_pos = u_pos.astype(jnp.int32)
  v_pos = v_pos.astype(jnp.int32)
  v_neg_t = v_neg.astype(jnp.int32).T
  src_lin = _to_rowmajor(src_w.T)
  tgt_lin = _to_rowmajor(tgt_w.T)
  partials = _sc_call(u_pos, v_pos,
                      v_neg_t[0], v_neg_t[1], v_neg_t[2], v_neg_t[3],
                      v_neg_t[4], src_lin, tgt_lin)
  return partials.sum() / jnp.float32(B_ * NNEG_)


# speedup vs baseline: 1.1468x; 1.1468x over previous
import jax
import jax.numpy as jnp
from jax import lax
from jax.experimental import pallas as pl
from jax.experimental.pallas import tpu as pltpu
from jax.experimental.pallas import tpu_sc as plsc

VOCAB_ = 1000000
DIM_ = 64
B_ = 16384
NNEG_ = 5
MARGIN_ = 1.0

NC = 2
NS = 16
NW = NC * NS
LANES = 16

BPW = B_ // NW
CHUNK = 128
NCHUNK = BPW // CHUNK
GROUPS = CHUNK // LANES


def _shuf(x, perm):
  return lax.gather(
      x, perm[:, None],
      lax.GatherDimensionNumbers(offset_dims=(), collapsed_slice_dims=(0,),
                                 start_index_map=(0,)),
      slice_sizes=(1,), mode=lax.GatherScatterMode.PROMISE_IN_BOUNDS)


def _sc_body(u_hbm, v_hbm, n0_hbm, n1_hbm, n2_hbm, n3_hbm, n4_hbm,
             src_hbm, tgt_hbm, out_hbm,
             iu, iv, in0, in1, in2, in3, in4,
             ru, rv, rn0, rn1, rn2, rn3, rn4,
             acc_v, sem):
  cid = lax.axis_index("c")
  sid = lax.axis_index("s")
  wid = cid * NS + sid

  lane = lax.iota(jnp.int32, LANES)
  perms = [lane ^ 1, lane ^ 2, lane ^ 4, lane ^ 8]
  total = jnp.zeros((LANES,), jnp.float32)

  for chunk in range(NCHUNK):
    base = wid * BPW + chunk * CHUNK
    pltpu.sync_copy(u_hbm.at[pl.ds(base, CHUNK)], iu)
    pltpu.sync_copy(v_hbm.at[pl.ds(base, CHUNK)], iv)
    pltpu.sync_copy(n0_hbm.at[pl.ds(base, CHUNK)], in0)
    pltpu.sync_copy(n1_hbm.at[pl.ds(base, CHUNK)], in1)
    pltpu.sync_copy(n2_hbm.at[pl.ds(base, CHUNK)], in2)
    pltpu.sync_copy(n3_hbm.at[pl.ds(base, CHUNK)], in3)
    pltpu.sync_copy(n4_hbm.at[pl.ds(base, CHUNK)], in4)

    def row_dma(g, carry):
      gbase = g * LANES
      sl = pl.ds(gbase, LANES)
      vu, vv = iu[sl], iv[sl]
      v0, v1, v2, v3, v4 = in0[sl], in1[sl], in2[sl], in3[sl], in4[sl]
      for j in range(LANES):
        e = gbase + j
        pltpu.make_async_copy(src_hbm.at[vu[j]], ru.at[e], sem).start()
        pltpu.make_async_copy(tgt_hbm.at[vv[j]], rv.at[e], sem).start()
        pltpu.make_async_copy(tgt_hbm.at[v0[j]], rn0.at[e], sem).start()
        pltpu.make_async_copy(tgt_hbm.at[v1[j]], rn1.at[e], sem).start()
        pltpu.make_async_copy(tgt_hbm.at[v2[j]], rn2.at[e], sem).start()
        pltpu.make_async_copy(tgt_hbm.at[v3[j]], rn3.at[e], sem).start()
        pltpu.make_async_copy(tgt_hbm.at[v4[j]], rn4.at[e], sem).start()
      return carry

    lax.fori_loop(0, GROUPS, row_dma, jnp.int32(0))
    for buf in (ru, rv, rn0, rn1, rn2, rn3, rn4):
      pltpu.make_async_copy(src_hbm.at[pl.ds(0, CHUNK)], buf, sem).wait()

    def elem_body(e, tot):
      pv = jnp.zeros((LANES,), jnp.float32)
      p0 = jnp.zeros((LANES,), jnp.float32)
      p1 = jnp.zeros((LANES,), jnp.float32)
      p2 = jnp.zeros((LANES,), jnp.float32)
      p3 = jnp.zeros((LANES,), jnp.float32)
      p4 = jnp.zeros((LANES,), jnp.float32)
      for k in range(DIM_ // LANES):
        sl = pl.ds(k * LANES, LANES)
        uc = ru[e, sl]
        pv = pv + uc * rv[e, sl]
        p0 = p0 + uc * rn0[e, sl]
        p1 = p1 + uc * rn1[e, sl]
        p2 = p2 + uc * rn2[e, sl]
        p3 = p3 + uc * rn3[e, sl]
        p4 = p4 + uc * rn4[e, sl]
      loss = jnp.zeros((LANES,), jnp.float32)
      for p in (p0, p1, p2, p3, p4):
        r = p - pv
        for perm in perms:
          r = r + _shuf(r, perm)
        loss = loss + jnp.maximum(r + MARGIN_, 0.0)
      return tot + loss

    total = total + lax.fori_loop(0, CHUNK, elem_body,
                                  jnp.zeros((LANES,), jnp.float32))

  acc_v[...] = jnp.where(lane == 0, total, jnp.float32(0.0))
  pltpu.sync_copy(acc_v, out_hbm.at[pl.ds(wid * LANES, LANES)])


@jax.jit
def _sc_call(u_pos, v_pos, n0, n1, n2, n3, n4, src_w, tgt_w):
  mesh = plsc.VectorSubcoreMesh(core_axis_name="c", subcore_axis_name="s")
  f = pl.kernel(
      _sc_body,
      out_type=jax.ShapeDtypeStruct((NW * LANES,), jnp.float32),
      mesh=mesh,
      scratch_types=[
          pltpu.VMEM((CHUNK,), jnp.int32),
          pltpu.VMEM((CHUNK,), jnp.int32),
          pltpu.VMEM((CHUNK,), jnp.int32),
          pltpu.VMEM((CHUNK,), jnp.int32),
          pltpu.VMEM((CHUNK,), jnp.int32),
          pltpu.VMEM((CHUNK,), jnp.int32),
          pltpu.VMEM((CHUNK,), jnp.int32),
          pltpu.VMEM((CHUNK, DIM_), jnp.float32),
          pltpu.VMEM((CHUNK, DIM_), jnp.float32),
          pltpu.VMEM((CHUNK, DIM_), jnp.float32),
          pltpu.VMEM((CHUNK, DIM_), jnp.float32),
          pltpu.VMEM((CHUNK, DIM_), jnp.float32),
          pltpu.VMEM((CHUNK, DIM_), jnp.float32),
          pltpu.VMEM((CHUNK, DIM_), jnp.float32),
          pltpu.VMEM((LANES,), jnp.float32),
          pltpu.SemaphoreType.DMA,
      ],
  )
  return f(u_pos, v_pos, n0, n1, n2, n3, n4, src_w, tgt_w)


def kernel(u_pos, v_pos, v_neg, src_w, tgt_w):
  u_pos = u_pos.astype(jnp.int32)
  v_pos = v_pos.astype(jnp.int32)
  v_neg_t = v_neg.astype(jnp.int32).T
  partials = _sc_call(u_pos, v_pos,
                      v_neg_t[0], v_neg_t[1], v_neg_t[2], v_neg_t[3],
                      v_neg_t[4], src_w, tgt_w)
  return partials.sum() / jnp.float32(B_ * NNEG_)
